# baseline (speedup 1.0000x reference)
import functools

import jax
import jax.numpy as jnp
from jax import lax
from jax.experimental import pallas as pl
from jax.experimental.pallas import tpu as pltpu
from jax.experimental.pallas import tpu_sc as plsc

NC = 2
NS = 16
NW = NC * NS


def _make_sc_agg(n, d, nblk):
  C = 128
  G = 40
  assert nblk % G == 0 and G % 2 == 0 and n % (8 * NS) == 0
  rpt = n // NS

  mesh = plsc.VectorSubcoreMesh(
      core_axis_name="c", subcore_axis_name="s", num_cores=NC,
      num_subcores=NS)

  scratch = [
      pltpu.VMEM((G, C), jnp.int32),
      pltpu.VMEM((G, C), jnp.int32),
      pltpu.VMEM((2, C, d), jnp.float32),
      pltpu.VMEM_SHARED((n, d), jnp.float32),
      pltpu.SemaphoreType.DMA,
      pltpu.SemaphoreType.DMA,
  ]

  def body(h_hbm, src_hbm, dst_hbm, zrows_hbm, agg_out,
           sidx_v, didx_v, rows_v, agg_sh, sem0, sem1):
    c = lax.axis_index("c")
    s = lax.axis_index("s")
    w = s * NC + c
    r0 = s * rpt

    pltpu.sync_copy(zrows_hbm, agg_sh.at[pl.ds(r0, rpt)])
    plsc.subcore_barrier()

    def group(q, carry):
      pltpu.sync_copy(src_hbm.at[w, pl.ds(q * G, G)], sidx_v)
      pltpu.sync_copy(dst_hbm.at[w, pl.ds(q * G, G)], didx_v)
      pltpu.async_copy(h_hbm.at[sidx_v.at[0]], rows_v.at[0], sem0)

      def pair(j, carry2):
        i0 = 2 * j
        i1 = i0 + 1
        pltpu.make_async_copy(h_hbm.at[sidx_v.at[i0]], rows_v.at[0],
                              sem0).wait()
        pltpu.async_copy(h_hbm.at[sidx_v.at[i1]], rows_v.at[1], sem1)
        pltpu.sync_copy(rows_v.at[0], agg_sh.at[didx_v.at[i0]], add=True)
        pltpu.make_async_copy(h_hbm.at[sidx_v.at[i1]], rows_v.at[1],
                              sem1).wait()

        @pl.when(j < G // 2 - 1)
        def _():
          pltpu.async_copy(h_hbm.at[sidx_v.at[i0 + 2]], rows_v.at[0], sem0)

        pltpu.sync_copy(rows_v.at[1], agg_sh.at[didx_v.at[i1]], add=True)
        return carry2

      lax.fori_loop(0, G // 2, pair, 0)
      return carry

    lax.fori_loop(0, nblk // G, group, 0)
    plsc.subcore_barrier()

    pltpu.sync_copy(agg_sh.at[pl.ds(r0, rpt)], agg_out.at[c, pl.ds(r0, rpt)])

  return pl.kernel(body, out_type=(jax.ShapeDtypeStruct((NC, n, d),
                                                        jnp.float32),),
                   mesh=mesh, scratch_types=scratch)


def _make_sc_deg(n, d, nblk):
  C = 128
  rpt = n // NS

  mesh = plsc.VectorSubcoreMesh(
      core_axis_name="c", subcore_axis_name="s", num_cores=NC,
      num_subcores=NS)

  scratch = [
      pltpu.VMEM((nblk, C), jnp.int32),
      pltpu.VMEM((C, d), jnp.float32),
      pltpu.VMEM_SHARED((n, d), jnp.float32),
      pltpu.SemaphoreType.DMA,
  ]

  def body(dst_hbm, ones_hbm, zrows_hbm, deg_out, didx_v, ones_v, deg_sh,
           sem):
    c = lax.axis_index("c")
    s = lax.axis_index("s")
    w = s * NC + c
    r0 = s * rpt

    pltpu.sync_copy(zrows_hbm, deg_sh.at[pl.ds(r0, rpt)])
    pltpu.sync_copy(ones_hbm, ones_v)
    pltpu.sync_copy(dst_hbm.at[w], didx_v)
    plsc.subcore_barrier()

    def fire(i, carry):
      pltpu.async_copy(ones_v, deg_sh.at[didx_v.at[i]], sem, add=True)
      return carry

    lax.fori_loop(0, nblk, fire, 0)

    def drain(i, carry):
      pltpu.make_async_copy(ones_v, deg_sh.at[didx_v.at[i]], sem).wait()
      return carry

    lax.fori_loop(0, nblk, drain, 0)
    plsc.subcore_barrier()
    pltpu.sync_copy(deg_sh.at[pl.ds(r0, rpt)], deg_out.at[c, pl.ds(r0, rpt)])

  return pl.kernel(body, out_type=(jax.ShapeDtypeStruct((NC, n, d),
                                                        jnp.float32),),
                   mesh=mesh, scratch_types=scratch)


def _mm_in_body(x_ref, w_ref, o_ref):
  o_ref[...] = jnp.dot(x_ref[...], w_ref[...],
                       preferred_element_type=jnp.float32)


def _layer_body(agg_ref, deg_ref, wc_ref, wl_ref, o_ref):
  d = jnp.maximum(deg_ref[0, :, 0:1] + deg_ref[1, :, 0:1], 1.0)
  h = (agg_ref[0] + agg_ref[1]) / d
  wcl = jnp.dot(wc_ref[...], wl_ref[...], preferred_element_type=jnp.float32)
  o_ref[...] = jnp.maximum(
      jnp.dot(h, wcl, preferred_element_type=jnp.float32), 0.0)


def _final_body(agg_ref, deg_ref, x0_ref, wc_ref, wl_ref, wt_ref, o_ref):
  d = jnp.maximum(deg_ref[0, :, 0:1] + deg_ref[1, :, 0:1], 1.0)
  h = (agg_ref[0] + agg_ref[1]) / d
  wcl = jnp.dot(wc_ref[...], wl_ref[...], preferred_element_type=jnp.float32)
  h2 = jnp.maximum(jnp.dot(h, wcl, preferred_element_type=jnp.float32), 0.0)
  o_ref[...] = jnp.maximum(
      jnp.dot(x0_ref[...] + h2, wt_ref[...],
              preferred_element_type=jnp.float32), 0.0)


def kernel(x, edge_index, W_in, W_conv0, W_conv1, W_lin0, W_lin1, W_trans):
  n0, d = x.shape
  e = edge_index.shape[1]
  n = (n0 // (8 * NS) + 1) * (8 * NS)
  x = jnp.pad(x, ((0, n - n0), (0, 0)))
  bn = n // 16
  grid = (16,)

  nblk = -(-e // (NW * 128 * 2)) * 2
  epad = NW * nblk * 128 - e
  src = jnp.concatenate(
      [edge_index[0], jnp.full((epad,), n0, jnp.int32)]).reshape(
          NW, nblk, 128)
  dst = jnp.concatenate(
      [edge_index[1],
       n0 + jnp.arange(epad, dtype=jnp.int32) % (n - n0)]).reshape(
           NW, nblk, 128)

  xspec = pl.BlockSpec((bn, d), lambda i: (i, 0))
  wspec = pl.BlockSpec((d, d), lambda i: (0, 0))
  aspec = pl.BlockSpec((NC, bn, d), lambda i: (0, i, 0))
  oshape = jax.ShapeDtypeStruct((n, d), jnp.float32)

  x0 = pl.pallas_call(
      _mm_in_body, grid=grid, in_specs=[xspec, wspec], out_specs=xspec,
      out_shape=oshape)(x, W_in)

  sc_agg = _make_sc_agg(n, d, nblk)
  sc_deg = _make_sc_deg(n, d, nblk)

  zrows = jnp.zeros((n // NS, d), jnp.float32)
  ones = jnp.ones((128, d), jnp.float32)

  (deg,) = sc_deg(dst, ones, zrows)
  (agg0,) = sc_agg(x0, src, dst, zrows)

  h1 = pl.pallas_call(
      _layer_body, grid=grid,
      in_specs=[aspec, aspec, wspec, wspec], out_specs=xspec,
      out_shape=oshape)(agg0, deg, W_conv0, W_lin0)

  (agg1,) = sc_agg(h1, src, dst, zrows)

  out = pl.pallas_call(
      _final_body, grid=grid,
      in_specs=[aspec, aspec, xspec, wspec, wspec, wspec], out_specs=xspec,
      out_shape=oshape)(agg1, deg, x0, W_conv1, W_lin1, W_trans)
  return out[:n0]


# speedup vs baseline: 2.9610x; 2.9610x over previous
import functools

import jax
import jax.numpy as jnp
from jax import lax
from jax.experimental import pallas as pl
from jax.experimental.pallas import tpu as pltpu
from jax.experimental.pallas import tpu_sc as plsc

NC = 2
NS = 16
NW = NC * NS


def _make_sc_agg(n, d, nblk):
  C = 128
  G = 40
  assert nblk % G == 0 and G % 2 == 0 and n % (8 * NS) == 0
  rpt = n // NS

  mesh = plsc.VectorSubcoreMesh(
      core_axis_name="c", subcore_axis_name="s", num_cores=NC,
      num_subcores=NS)

  scratch = [
      pltpu.VMEM((G, C), jnp.int32),
      pltpu.VMEM((G, C), jnp.int32),
      pltpu.VMEM((2, C, d), jnp.float32),
      pltpu.VMEM_SHARED((n, d), jnp.float32),
      pltpu.SemaphoreType.DMA,
      pltpu.SemaphoreType.DMA,
  ]

  def body(h_hbm, src_hbm, dst_hbm, zrows_hbm, agg_out,
           sidx_v, didx_v, rows_v, agg_sh, sem0, sem1):
    c = lax.axis_index("c")
    s = lax.axis_index("s")
    w = s * NC + c
    r0 = s * rpt

    pltpu.sync_copy(zrows_hbm, agg_sh.at[pl.ds(r0, rpt)])
    plsc.subcore_barrier()

    def group(q, carry):
      pltpu.sync_copy(src_hbm.at[w, pl.ds(q * G, G)], sidx_v)
      pltpu.sync_copy(dst_hbm.at[w, pl.ds(q * G, G)], didx_v)
      pltpu.async_copy(h_hbm.at[sidx_v.at[0]], rows_v.at[0], sem0)

      def pair(j, carry2):
        i0 = 2 * j
        i1 = i0 + 1
        pltpu.make_async_copy(h_hbm.at[sidx_v.at[i0]], rows_v.at[0],
                              sem0).wait()
        pltpu.async_copy(h_hbm.at[sidx_v.at[i1]], rows_v.at[1], sem1)
        pltpu.sync_copy(rows_v.at[0], agg_sh.at[didx_v.at[i0]], add=True)
        pltpu.make_async_copy(h_hbm.at[sidx_v.at[i1]], rows_v.at[1],
                              sem1).wait()

        @pl.when(j < G // 2 - 1)
        def _():
          pltpu.async_copy(h_hbm.at[sidx_v.at[i0 + 2]], rows_v.at[0], sem0)

        pltpu.sync_copy(rows_v.at[1], agg_sh.at[didx_v.at[i1]], add=True)
        return carry2

      lax.fori_loop(0, G // 2, pair, 0)
      return carry

    lax.fori_loop(0, nblk // G, group, 0)
    plsc.subcore_barrier()

    pltpu.sync_copy(agg_sh.at[pl.ds(r0, rpt)], agg_out.at[c, pl.ds(r0, rpt)])

  return pl.kernel(body, out_type=(jax.ShapeDtypeStruct((NC, n, d),
                                                        jnp.float32),),
                   mesh=mesh, scratch_types=scratch)


def _make_sc_deg(n, d, nblk):
  C = 128
  rpt = n // NS

  mesh = plsc.VectorSubcoreMesh(
      core_axis_name="c", subcore_axis_name="s", num_cores=NC,
      num_subcores=NS)

  scratch = [
      pltpu.VMEM((nblk, C), jnp.int32),
      pltpu.VMEM((C, d), jnp.float32),
      pltpu.VMEM_SHARED((n, d), jnp.float32),
      pltpu.SemaphoreType.DMA,
  ]

  def body(dst_hbm, ones_hbm, zrows_hbm, deg_out, didx_v, ones_v, deg_sh,
           sem):
    c = lax.axis_index("c")
    s = lax.axis_index("s")
    w = s * NC + c
    r0 = s * rpt

    pltpu.sync_copy(zrows_hbm, deg_sh.at[pl.ds(r0, rpt)])
    pltpu.sync_copy(ones_hbm, ones_v)
    pltpu.sync_copy(dst_hbm.at[w], didx_v)
    plsc.subcore_barrier()

    def fire(i, carry):
      pltpu.async_copy(ones_v, deg_sh.at[didx_v.at[i]], sem, add=True)
      return carry

    lax.fori_loop(0, nblk, fire, 0)

    def drain(i, carry):
      pltpu.make_async_copy(ones_v, deg_sh.at[didx_v.at[i]], sem).wait()
      return carry

    lax.fori_loop(0, nblk, drain, 0)
    plsc.subcore_barrier()
    pltpu.sync_copy(deg_sh.at[pl.ds(r0, rpt)], deg_out.at[c, pl.ds(r0, rpt)])

  return pl.kernel(body, out_type=(jax.ShapeDtypeStruct((NC, n, d),
                                                        jnp.float32),),
                   mesh=mesh, scratch_types=scratch)


def _mm_in_body(x_ref, w_ref, o_ref):
  o_ref[...] = jnp.dot(x_ref[...], w_ref[...],
                       preferred_element_type=jnp.float32)


def _layer_body(agg_ref, deg_ref, wc_ref, wl_ref, o_ref):
  d = jnp.maximum(deg_ref[0, :, 0:1] + deg_ref[1, :, 0:1], 1.0)
  h = (agg_ref[0] + agg_ref[1]) / d
  wcl = jnp.dot(wc_ref[...], wl_ref[...], preferred_element_type=jnp.float32)
  o_ref[...] = jnp.maximum(
      jnp.dot(h, wcl, preferred_element_type=jnp.float32), 0.0)


def _final_body(agg_ref, deg_ref, x0_ref, wc_ref, wl_ref, wt_ref, o_ref):
  d = jnp.maximum(deg_ref[0, :, 0:1] + deg_ref[1, :, 0:1], 1.0)
  h = (agg_ref[0] + agg_ref[1]) / d
  wcl = jnp.dot(wc_ref[...], wl_ref[...], preferred_element_type=jnp.float32)
  h2 = jnp.maximum(jnp.dot(h, wcl, preferred_element_type=jnp.float32), 0.0)
  o_ref[...] = jnp.maximum(
      jnp.dot(x0_ref[...] + h2, wt_ref[...],
              preferred_element_type=jnp.float32), 0.0)


def kernel(x, edge_index, W_in, W_conv0, W_conv1, W_lin0, W_lin1, W_trans):
  n0, d = x.shape
  e = edge_index.shape[1]
  n = (n0 // (8 * NS) + 1) * (8 * NS)
  x = jnp.pad(x, ((0, n - n0), (0, 0)))
  bn = n // 16
  grid = (16,)

  nblk = -(-e // (NW * 128 * 2)) * 2
  epad = NW * nblk * 128 - e
  pad_ar = jnp.arange(epad, dtype=jnp.int32)
  src = jnp.concatenate(
      [edge_index[0], pad_ar % n0]).reshape(NW, nblk, 128)
  dst = jnp.concatenate(
      [edge_index[1], n0 + pad_ar % (n - n0)]).reshape(NW, nblk, 128)

  xspec = pl.BlockSpec((bn, d), lambda i: (i, 0))
  wspec = pl.BlockSpec((d, d), lambda i: (0, 0))
  aspec = pl.BlockSpec((NC, bn, d), lambda i: (0, i, 0))
  oshape = jax.ShapeDtypeStruct((n, d), jnp.float32)

  x0 = pl.pallas_call(
      _mm_in_body, grid=grid, in_specs=[xspec, wspec], out_specs=xspec,
      out_shape=oshape)(x, W_in)

  sc_agg = _make_sc_agg(n, d, nblk)
  sc_deg = _make_sc_deg(n, d, nblk)

  zrows = jnp.zeros((n // NS, d), jnp.float32)
  ones = jnp.ones((128, d), jnp.float32)

  (deg,) = sc_deg(dst, ones, zrows)
  (agg0,) = sc_agg(x0, src, dst, zrows)

  h1 = pl.pallas_call(
      _layer_body, grid=grid,
      in_specs=[aspec, aspec, wspec, wspec], out_specs=xspec,
      out_shape=oshape)(agg0, deg, W_conv0, W_lin0)

  (agg1,) = sc_agg(h1, src, dst, zrows)

  out = pl.pallas_call(
      _final_body, grid=grid,
      in_specs=[aspec, aspec, xspec, wspec, wspec, wspec], out_specs=xspec,
      out_shape=oshape)(agg1, deg, x0, W_conv1, W_lin1, W_trans)
  return out[:n0]


# speedup vs baseline: 3.0201x; 1.0200x over previous
import jax
import jax.numpy as jnp
import numpy as np
from jax import lax
from jax.experimental import pallas as pl
from jax.experimental.pallas import tpu as pltpu
from jax.experimental.pallas import tpu_sc as plsc

NC = 2
NS = 16
NW = NC * NS


def _make_sc_agg(n, d, nblk):
  C = 128
  G = 40
  assert nblk % G == 0 and G % 2 == 0 and n % (8 * NS) == 0
  rpt = n // NS

  mesh = plsc.VectorSubcoreMesh(
      core_axis_name="c", subcore_axis_name="s", num_cores=NC,
      num_subcores=NS)

  scratch = [
      pltpu.VMEM((G, C), jnp.int32),
      pltpu.VMEM((G, C), jnp.int32),
      pltpu.VMEM((2, C, d), jnp.float32),
      pltpu.VMEM_SHARED((n, d), jnp.float32),
      pltpu.SemaphoreType.DMA,
      pltpu.SemaphoreType.DMA,
  ]

  def body(h_hbm, src_hbm, dst_hbm, zrows_hbm, agg_out,
           sidx_v, didx_v, rows_v, agg_sh, sem0, sem1):
    c = lax.axis_index("c")
    s = lax.axis_index("s")
    w = s * NC + c
    r0 = s * rpt

    pltpu.sync_copy(zrows_hbm, agg_sh.at[pl.ds(r0, rpt)])
    plsc.subcore_barrier()

    def group(q, carry):
      pltpu.sync_copy(src_hbm.at[w, pl.ds(q * G, G)], sidx_v)
      pltpu.sync_copy(dst_hbm.at[w, pl.ds(q * G, G)], didx_v)
      pltpu.async_copy(h_hbm.at[sidx_v.at[0]], rows_v.at[0], sem0)

      def pair(j, carry2):
        i0 = 2 * j
        i1 = i0 + 1
        pltpu.make_async_copy(h_hbm.at[sidx_v.at[i0]], rows_v.at[0],
                              sem0).wait()
        pltpu.async_copy(h_hbm.at[sidx_v.at[i1]], rows_v.at[1], sem1)
        pltpu.sync_copy(rows_v.at[0], agg_sh.at[didx_v.at[i0]], add=True)
        pltpu.make_async_copy(h_hbm.at[sidx_v.at[i1]], rows_v.at[1],
                              sem1).wait()

        @pl.when(j < G // 2 - 1)
        def _():
          pltpu.async_copy(h_hbm.at[sidx_v.at[i0 + 2]], rows_v.at[0], sem0)

        pltpu.sync_copy(rows_v.at[1], agg_sh.at[didx_v.at[i1]], add=True)
        return carry2

      lax.fori_loop(0, G // 2, pair, 0)
      return carry

    lax.fori_loop(0, nblk // G, group, 0)
    plsc.subcore_barrier()

    pltpu.sync_copy(agg_sh.at[pl.ds(r0, rpt)], agg_out.at[c, pl.ds(r0, rpt)])

  return pl.kernel(body, out_type=(jax.ShapeDtypeStruct((NC, n, d),
                                                        jnp.float32),),
                   mesh=mesh, scratch_types=scratch)


def _make_sc_deg(n, d, nblk):
  C = 128
  rpt = n // NS

  mesh = plsc.VectorSubcoreMesh(
      core_axis_name="c", subcore_axis_name="s", num_cores=NC,
      num_subcores=NS)

  scratch = [
      pltpu.VMEM((nblk, C), jnp.int32),
      pltpu.VMEM((C, d), jnp.float32),
      pltpu.VMEM_SHARED((n, d), jnp.float32),
      pltpu.SemaphoreType.DMA,
  ]

  def body(dst_hbm, ones_hbm, zrows_hbm, deg_out, didx_v, ones_v, deg_sh,
           sem):
    c = lax.axis_index("c")
    s = lax.axis_index("s")
    w = s * NC + c
    r0 = s * rpt

    pltpu.sync_copy(zrows_hbm, deg_sh.at[pl.ds(r0, rpt)])
    pltpu.sync_copy(ones_hbm, ones_v)
    pltpu.sync_copy(dst_hbm.at[w], didx_v)
    plsc.subcore_barrier()

    def fire(i, carry):
      pltpu.async_copy(ones_v, deg_sh.at[didx_v.at[i]], sem, add=True)
      return carry

    lax.fori_loop(0, nblk, fire, 0)

    def drain(i, carry):
      pltpu.make_async_copy(ones_v, deg_sh.at[didx_v.at[i]], sem).wait()
      return carry

    lax.fori_loop(0, nblk, drain, 0)
    plsc.subcore_barrier()
    pltpu.sync_copy(deg_sh.at[pl.ds(r0, rpt)], deg_out.at[c, pl.ds(r0, rpt)])

  return pl.kernel(body, out_type=(jax.ShapeDtypeStruct((NC, n, d),
                                                        jnp.float32),),
                   mesh=mesh, scratch_types=scratch)


def _mm_in_body(x_ref, w_ref, o_ref):
  o_ref[...] = jnp.dot(x_ref[...], w_ref[...],
                       preferred_element_type=jnp.float32)


def _layer_body(agg_ref, deg_ref, wc_ref, wl_ref, o_ref):
  d = jnp.maximum(deg_ref[0, :, 0:1] + deg_ref[1, :, 0:1], 1.0)
  h = (agg_ref[0] + agg_ref[1]) / d
  wcl = jnp.dot(wc_ref[...], wl_ref[...], preferred_element_type=jnp.float32)
  o_ref[...] = jnp.maximum(
      jnp.dot(h, wcl, preferred_element_type=jnp.float32), 0.0)


def _final_body(agg_ref, deg_ref, x0_ref, wc_ref, wl_ref, wt_ref, o_ref):
  d = jnp.maximum(deg_ref[0, :, 0:1] + deg_ref[1, :, 0:1], 1.0)
  h = (agg_ref[0] + agg_ref[1]) / d
  wcl = jnp.dot(wc_ref[...], wl_ref[...], preferred_element_type=jnp.float32)
  h2 = jnp.maximum(jnp.dot(h, wcl, preferred_element_type=jnp.float32), 0.0)
  o_ref[...] = jnp.maximum(
      jnp.dot(x0_ref[...] + h2, wt_ref[...],
              preferred_element_type=jnp.float32), 0.0)


def kernel(x, edge_index, W_in, W_conv0, W_conv1, W_lin0, W_lin1, W_trans):
  n0, d = x.shape
  e = edge_index.shape[1]
  n = (n0 // (8 * NS) + 1) * (8 * NS)
  x = jnp.pad(x, ((0, n - n0), (0, 0)))
  bn = n // 16
  grid = (16,)

  nblk = -(-e // (NW * 128 * 2)) * 2
  epad = NW * nblk * 128 - e
  pad_ar = np.arange(epad, dtype=np.int32)
  src = jnp.concatenate(
      [edge_index[0], jnp.asarray(pad_ar % n0)]).reshape(NW, nblk, 128)
  dst = jnp.concatenate(
      [edge_index[1], jnp.asarray(n0 + pad_ar % (n - n0))]).reshape(
          NW, nblk, 128)

  xspec = pl.BlockSpec((bn, d), lambda i: (i, 0))
  wspec = pl.BlockSpec((d, d), lambda i: (0, 0))
  aspec = pl.BlockSpec((NC, bn, d), lambda i: (0, i, 0))
  dspec = pl.BlockSpec((NC, bn, d), lambda i: (0, i, 0))
  oshape = jax.ShapeDtypeStruct((n, d), jnp.float32)

  x0 = pl.pallas_call(
      _mm_in_body, grid=grid, in_specs=[xspec, wspec], out_specs=xspec,
      out_shape=oshape)(x, W_in)

  sc_agg = _make_sc_agg(n, d, nblk)
  sc_deg = _make_sc_deg(n, d, nblk)

  zrows = jnp.zeros((n // NS, d), jnp.float32)
  ones = jnp.ones((128, d), jnp.float32)

  (deg,) = sc_deg(dst, ones, zrows)
  (agg0,) = sc_agg(x0, src, dst, zrows)

  h1 = pl.pallas_call(
      _layer_body, grid=grid,
      in_specs=[aspec, dspec, wspec, wspec], out_specs=xspec,
      out_shape=oshape)(agg0, deg, W_conv0, W_lin0)

  (agg1,) = sc_agg(h1, src, dst, zrows)

  bn0 = n0 // 10
  out = pl.pallas_call(
      _final_body, grid=(10,),
      in_specs=[pl.BlockSpec((NC, bn0, d), lambda i: (0, i, 0)),
                pl.BlockSpec((NC, bn0, d), lambda i: (0, i, 0)),
                pl.BlockSpec((bn0, d), lambda i: (i, 0)),
                wspec, wspec, wspec],
      out_specs=pl.BlockSpec((bn0, d), lambda i: (i, 0)),
      out_shape=jax.ShapeDtypeStruct((n0, d), jnp.float32))(
          agg1, deg, x0, W_conv1, W_lin1, W_trans)
  return out
